# baseline (speedup 1.0000x reference)
import functools

import jax
import jax.numpy as jnp
from jax import lax
from jax.experimental import pallas as pl
from jax.experimental.pallas import tpu as pltpu
from jax.experimental.pallas import tpu_sc as plsc

NUM_USERS = 8000
NUM_POIS = 2000
N_NODES = 10000
N_PAD = 10240
DIM = 128
POI_DIM = 256
N_EDGES = 320000
BATCH = 4096

NC = 2
NS = 16
NW = NC * NS
LANES = 16

CHUNK = 100
NCHUNK = 100
EPW = NCHUNK * CHUNK
ROWS_PER_TILE = N_PAD // NS
OUT_STEP = 64
NOUT = ROWS_PER_TILE // OUT_STEP
BPW = BATCH // NW
BPT = BATCH // NS

_MESH = plsc.VectorSubcoreMesh(core_axis_name="c", subcore_axis_name="s")
_SC_PARAMS = pltpu.CompilerParams(needs_layout_passes=False)


@functools.partial(
    pl.kernel,
    out_type=jax.ShapeDtypeStruct((NW * N_NODES,), jnp.float32),
    mesh=_MESH,
    scratch_types=[
        pltpu.VMEM((EPW,), jnp.int32),
        pltpu.VMEM((N_NODES,), jnp.float32),
    ],
    compiler_params=_SC_PARAMS,
)
def _deg_kernel(dst_hbm, degp_hbm, idx_v, deg_v):
    c = lax.axis_index("c")
    s = lax.axis_index("s")
    wid = s * NC + c

    zeros16 = jnp.zeros((LANES,), jnp.float32)

    def zero_body(i, carry):
        deg_v[pl.ds(i * LANES, LANES)] = zeros16
        return carry

    lax.fori_loop(0, N_NODES // LANES, zero_body, 0)

    pltpu.sync_copy(dst_hbm.at[pl.ds(wid * EPW, EPW)], idx_v)

    ones16 = jnp.ones((LANES,), jnp.float32)

    def hist_body(i, carry):
        idx = idx_v[pl.ds(i * LANES, LANES)]
        plsc.addupdate_scatter(deg_v, [idx], ones16)
        return carry

    lax.fori_loop(0, EPW // LANES, hist_body, 0)

    pltpu.sync_copy(deg_v, degp_hbm.at[pl.ds(wid * N_NODES, N_NODES)])


@functools.partial(
    pl.kernel,
    out_type=(
        jax.ShapeDtypeStruct((N_PAD, DIM), jnp.float32),
        jax.ShapeDtypeStruct((N_PAD, DIM), jnp.float32),
    ),
    mesh=_MESH,
    scratch_types=[
        pltpu.VMEM((NCHUNK, CHUNK), jnp.int32),
        pltpu.VMEM((1, CHUNK), jnp.int32),
        pltpu.VMEM((1, CHUNK), jnp.int32),
        pltpu.VMEM((CHUNK, DIM), jnp.float32),
        pltpu.VMEM((CHUNK, DIM), jnp.float32),
        pltpu.VMEM_SHARED((N_PAD, DIM), jnp.float32),
        pltpu.SemaphoreType.DMA,
        pltpu.SemaphoreType.DMA,
        pltpu.SemaphoreType.DMA,
        pltpu.SemaphoreType.DMA,
    ],
    compiler_params=_SC_PARAMS,
)
def _agg_kernel(g_hbm, src_hbm, dst_hbm, zeros_hbm, out_a, out_b,
                sidx_v, didx_a, didx_b, rows_a, rows_b, acc_sh,
                sem_a, sem_b, sem_da, sem_db):
    c = lax.axis_index("c")
    s = lax.axis_index("s")
    wid = s * NC + c

    zbuf = rows_a.at[pl.ds(0, OUT_STEP)]
    pltpu.sync_copy(zeros_hbm, zbuf)
    zdescs = [
        pltpu.async_copy(
            zbuf,
            acc_sh.at[pl.ds(s * ROWS_PER_TILE + t * OUT_STEP, OUT_STEP)],
            sem_a)
        for t in range(NOUT)
    ]
    pltpu.sync_copy(src_hbm.at[wid], sidx_v)
    for d in zdescs:
        d.wait()

    plsc.subcore_barrier()

    dst_w = dst_hbm.at[wid]

    pltpu.async_copy(g_hbm.at[sidx_v.at[0]], rows_a, sem_a)
    pltpu.async_copy(dst_w.at[0], didx_a, sem_da)

    def edge_body(i, carry):
        j = 2 * i
        pltpu.async_copy(g_hbm.at[sidx_v.at[j + 1]], rows_b, sem_b)
        pltpu.async_copy(dst_w.at[j + 1], didx_b, sem_db)
        pltpu.make_async_copy(g_hbm.at[sidx_v.at[0]], rows_a, sem_a).wait()
        pltpu.make_async_copy(dst_w.at[0], didx_a, sem_da).wait()
        pltpu.sync_copy(rows_a, acc_sh.at[didx_a.at[0]], add=True)

        @pl.when(j + 2 < NCHUNK)
        def _():
            pltpu.async_copy(g_hbm.at[sidx_v.at[j + 2]], rows_a, sem_a)
            pltpu.async_copy(dst_w.at[j + 2], didx_a, sem_da)

        pltpu.make_async_copy(g_hbm.at[sidx_v.at[0]], rows_b, sem_b).wait()
        pltpu.make_async_copy(dst_w.at[0], didx_b, sem_db).wait()
        pltpu.sync_copy(rows_b, acc_sh.at[didx_b.at[0]], add=True)
        return carry

    lax.fori_loop(0, NCHUNK // 2, edge_body, 0)

    plsc.subcore_barrier()

    odescs = [None] * NOUT
    for t in range(NOUT):
        full, sem_t = (rows_a, sem_a) if t % 2 == 0 else (rows_b, sem_b)
        buf = full.at[pl.ds(0, OUT_STEP)]
        if t >= 2:
            odescs[t - 2].wait()
        r = s * ROWS_PER_TILE + t * OUT_STEP
        pltpu.sync_copy(acc_sh.at[pl.ds(r, OUT_STEP)], buf)

        @pl.when(c == 0)
        def _():
            pltpu.async_copy(buf, out_a.at[pl.ds(r, OUT_STEP)], sem_t)

        @pl.when(c == 1)
        def _():
            pltpu.async_copy(buf, out_b.at[pl.ds(r, OUT_STEP)], sem_t)

        odescs[t] = pltpu.make_async_copy(
            buf, out_a.at[pl.ds(r, OUT_STEP)], sem_t)
    odescs[NOUT - 2].wait()
    odescs[NOUT - 1].wait()


@functools.partial(
    pl.kernel,
    out_type=(
        jax.ShapeDtypeStruct((BATCH, DIM), jnp.float32),
        jax.ShapeDtypeStruct((BATCH, DIM), jnp.float32),
        jax.ShapeDtypeStruct((BATCH, DIM), jnp.float32),
        jax.ShapeDtypeStruct((BATCH, DIM), jnp.float32),
    ),
    mesh=_MESH,
    scratch_types=[
        pltpu.VMEM((NCHUNK, CHUNK), jnp.int32),
        pltpu.VMEM((1, CHUNK), jnp.int32),
        pltpu.VMEM((1, CHUNK), jnp.int32),
        pltpu.VMEM((CHUNK, DIM), jnp.float32),
        pltpu.VMEM((CHUNK, DIM), jnp.float32),
        pltpu.VMEM((BPT,), jnp.int32),
        pltpu.VMEM_SHARED((N_PAD, DIM), jnp.float32),
        pltpu.SemaphoreType.DMA,
        pltpu.SemaphoreType.DMA,
        pltpu.SemaphoreType.DMA,
        pltpu.SemaphoreType.DMA,
    ],
    compiler_params=_SC_PARAMS,
)
def _agg_gather_kernel(g_hbm, src_hbm, dst_hbm, zeros_hbm, ut, dinv128, uidx,
                       y1a, y1b, yu, yd,
                       sidx_v, didx_a, didx_b, rows_a, rows_b, uidx_v, acc_sh,
                       sem_a, sem_b, sem_da, sem_db):
    c = lax.axis_index("c")
    s = lax.axis_index("s")
    wid = s * NC + c

    zbuf = rows_a.at[pl.ds(0, OUT_STEP)]
    pltpu.sync_copy(zeros_hbm, zbuf)
    zdescs = [
        pltpu.async_copy(
            zbuf,
            acc_sh.at[pl.ds(s * ROWS_PER_TILE + t * OUT_STEP, OUT_STEP)],
            sem_a)
        for t in range(NOUT)
    ]
    pltpu.sync_copy(src_hbm.at[wid], sidx_v)
    pltpu.sync_copy(uidx.at[pl.ds(s * BPT, BPT)], uidx_v)
    for d in zdescs:
        d.wait()

    plsc.subcore_barrier()

    dst_w = dst_hbm.at[wid]

    pltpu.async_copy(g_hbm.at[sidx_v.at[0]], rows_a, sem_a)
    pltpu.async_copy(dst_w.at[0], didx_a, sem_da)

    def edge_body(i, carry):
        j = 2 * i
        pltpu.async_copy(g_hbm.at[sidx_v.at[j + 1]], rows_b, sem_b)
        pltpu.async_copy(dst_w.at[j + 1], didx_b, sem_db)
        pltpu.make_async_copy(g_hbm.at[sidx_v.at[0]], rows_a, sem_a).wait()
        pltpu.make_async_copy(dst_w.at[0], didx_a, sem_da).wait()
        pltpu.sync_copy(rows_a, acc_sh.at[didx_a.at[0]], add=True)

        @pl.when(j + 2 < NCHUNK)
        def _():
            pltpu.async_copy(g_hbm.at[sidx_v.at[j + 2]], rows_a, sem_a)
            pltpu.async_copy(dst_w.at[j + 2], didx_a, sem_da)

        pltpu.make_async_copy(g_hbm.at[sidx_v.at[0]], rows_b, sem_b).wait()
        pltpu.make_async_copy(dst_w.at[0], didx_b, sem_db).wait()
        pltpu.sync_copy(rows_b, acc_sh.at[didx_b.at[0]], add=True)
        return carry

    lax.fori_loop(0, NCHUNK // 2, edge_body, 0)

    plsc.subcore_barrier()

    GSTEP = 64
    gbuf = rows_a.at[pl.ds(0, GSTEP)]
    for k in range(BPT // GSTEP):
        gslice = uidx_v.at[pl.ds(k * GSTEP, GSTEP)]
        pltpu.async_copy(acc_sh.at[gslice], gbuf, sem_a).wait()

        @pl.when(c == 0)
        def _():
            pltpu.async_copy(g_hbm.at[gslice], gbuf, sem_a, add=True).wait()
            pltpu.sync_copy(gbuf, y1a.at[pl.ds(s * BPT + k * GSTEP, GSTEP)])

        @pl.when(c == 1)
        def _():
            pltpu.sync_copy(gbuf, y1b.at[pl.ds(s * BPT + k * GSTEP, GSTEP)])

    hbuf = rows_b.at[pl.ds(0, GSTEP)]
    for tbl, outy in ((ut, yu), (dinv128, yd)):
        for k in range(BPW // GSTEP):
            off = c * BPW + k * GSTEP
            hslice = uidx_v.at[pl.ds(off, GSTEP)]
            pltpu.async_copy(tbl.at[hslice], hbuf, sem_b).wait()
            pltpu.sync_copy(
                hbuf, outy.at[pl.ds(wid * BPW + k * GSTEP, GSTEP)])


def _dense1a_body(ut_ref, poi_ref, Wp_ref, bp_ref, Wg1_ref, h1_ref):
    hp = jnp.dot(poi_ref[...], Wp_ref[...],
                 preferred_element_type=jnp.float32) + bp_ref[...]
    h1_ref[0:NUM_USERS, :] = jnp.dot(ut_ref[...], Wg1_ref[...],
                                     preferred_element_type=jnp.float32)
    h1_ref[NUM_USERS:N_NODES, :] = jnp.dot(
        hp, Wg1_ref[...], preferred_element_type=jnp.float32)
    h1_ref[N_NODES:N_PAD, :] = jnp.zeros((N_PAD - N_NODES, DIM), jnp.float32)


_dense1a = pl.pallas_call(
    _dense1a_body,
    out_shape=jax.ShapeDtypeStruct((N_PAD, DIM), jnp.float32),
)


def _dense1b_body(h1_ref, degp_ref, g1_ref, dinv16_ref, dinv128_ref):
    deg = jnp.sum(degp_ref[...], axis=0) + 1.0
    dinv = lax.rsqrt(deg)
    g1_ref[0:N_NODES, :] = h1_ref[0:N_NODES, :] * dinv[:, None]
    g1_ref[N_NODES:N_PAD, :] = jnp.zeros((N_PAD - N_NODES, DIM), jnp.float32)
    dinv16_ref[0:N_NODES, :] = jnp.broadcast_to(dinv[:, None],
                                                (N_NODES, LANES))
    dinv16_ref[N_NODES:N_PAD, :] = jnp.ones((N_PAD - N_NODES, LANES),
                                            jnp.float32)
    dinv128_ref[0:N_NODES, :] = jnp.broadcast_to(dinv[:, None],
                                                 (N_NODES, DIM))
    dinv128_ref[N_NODES:N_PAD, :] = jnp.ones((N_PAD - N_NODES, DIM),
                                             jnp.float32)


_dense1b = pl.pallas_call(
    _dense1b_body,
    out_shape=(
        jax.ShapeDtypeStruct((N_PAD, DIM), jnp.float32),
        jax.ShapeDtypeStruct((N_PAD, LANES), jnp.float32),
        jax.ShapeDtypeStruct((N_PAD, DIM), jnp.float32),
    ),
)


def _dense2_body(sa_ref, sb_ref, g1_ref, dinv16_ref, bg1_ref, Wg2_ref,
                 g2_ref):
    dinv = dinv16_ref[:, 0:1]
    t = dinv * (sa_ref[...] + sb_ref[...] + g1_ref[...]) + bg1_ref[...]
    x1 = jnp.maximum(t, 0.2 * t)
    g2_ref[...] = jnp.dot(x1, Wg2_ref[...],
                          preferred_element_type=jnp.float32) * dinv


_dense2 = pl.pallas_call(
    _dense2_body,
    out_shape=jax.ShapeDtypeStruct((N_PAD, DIM), jnp.float32),
)


def _final_body(y1a_ref, y1b_ref, yu_ref, yd_ref, bg2_ref, Wf_ref, bf_ref,
                out_ref):
    t = yd_ref[...] * (y1a_ref[...] + y1b_ref[...]) + bg2_ref[...]
    x2 = jnp.maximum(t, 0.2 * t)
    out_ref[...] = jnp.dot(x2 + yu_ref[...], Wf_ref[...],
                           preferred_element_type=jnp.float32) + bf_ref[...]


_final = pl.pallas_call(
    _final_body,
    out_shape=jax.ShapeDtypeStruct((BATCH, DIM), jnp.float32),
)


@jax.jit
def kernel(user_idx, poi_embeddings, edge_index, user_table,
           Wp, bp, Wg1, bg1, Wg2, bg2, Wf, bf):
    src = edge_index[0].astype(jnp.int32)
    dst = edge_index[1].astype(jnp.int32)
    src3 = src.reshape(NW, NCHUNK, CHUNK)
    dst4 = dst.reshape(NW, NCHUNK, 1, CHUNK)
    zeros = jnp.zeros((OUT_STEP, DIM), jnp.float32)
    uidx = user_idx.astype(jnp.int32)

    degp = _deg_kernel(dst).reshape(NW, N_NODES)
    h1 = _dense1a(user_table, poi_embeddings, Wp, bp, Wg1)
    g1, dinv16, dinv128 = _dense1b(h1, degp)
    s1a, s1b = _agg_kernel(g1, src3, dst4, zeros)
    g2 = _dense2(s1a, s1b, g1, dinv16, bg1, Wg2)
    y1a, y1b, yu, yd = _agg_gather_kernel(g2, src3, dst4, zeros,
                                          user_table, dinv128, uidx)
    return _final(y1a, y1b, yu, yd, bg2, Wf, bf)


# speedup vs baseline: 1.0289x; 1.0289x over previous
import functools

import jax
import jax.numpy as jnp
from jax import lax
from jax.experimental import pallas as pl
from jax.experimental.pallas import tpu as pltpu
from jax.experimental.pallas import tpu_sc as plsc

NUM_USERS = 8000
NUM_POIS = 2000
N_NODES = 10000
N_PAD = 10240
DIM = 128
POI_DIM = 256
N_EDGES = 320000
BATCH = 4096

NC = 2
NS = 16
NW = NC * NS
LANES = 16

EPW = N_EDGES // NW
CHUNK = 128
NFULL = EPW // CHUNK
TAIL = EPW - NFULL * CHUNK
ROWS_PER_TILE = N_PAD // NS
OUT_STEP = 128
NOUT = ROWS_PER_TILE // OUT_STEP
BPW = BATCH // NW
BPT = BATCH // NS

_MESH = plsc.VectorSubcoreMesh(core_axis_name="c", subcore_axis_name="s")
_SC_PARAMS = pltpu.CompilerParams(needs_layout_passes=False)


@functools.partial(
    pl.kernel,
    out_type=jax.ShapeDtypeStruct((NW * N_NODES,), jnp.float32),
    mesh=_MESH,
    scratch_types=[
        pltpu.VMEM((EPW,), jnp.int32),
        pltpu.VMEM((N_NODES,), jnp.float32),
    ],
    compiler_params=_SC_PARAMS,
)
def _deg_kernel(dst_hbm, degp_hbm, idx_v, deg_v):
    c = lax.axis_index("c")
    s = lax.axis_index("s")
    wid = s * NC + c

    zeros16 = jnp.zeros((LANES,), jnp.float32)

    def zero_body(i, carry):
        deg_v[pl.ds(i * LANES, LANES)] = zeros16
        return carry

    lax.fori_loop(0, N_NODES // LANES, zero_body, 0)

    pltpu.sync_copy(dst_hbm.at[pl.ds(wid * EPW, EPW)], idx_v)

    ones16 = jnp.ones((LANES,), jnp.float32)

    def hist_body(i, carry):
        idx = idx_v[pl.ds(i * LANES, LANES)]
        plsc.addupdate_scatter(deg_v, [idx], ones16)
        return carry

    lax.fori_loop(0, EPW // LANES, hist_body, 0)

    pltpu.sync_copy(deg_v, degp_hbm.at[pl.ds(wid * N_NODES, N_NODES)])


_AGG_SCRATCH = [
    pltpu.VMEM((EPW,), jnp.int32),
    pltpu.VMEM((CHUNK,), jnp.int32),
    pltpu.VMEM((CHUNK,), jnp.int32),
    pltpu.VMEM((TAIL,), jnp.int32),
    pltpu.VMEM((CHUNK, DIM), jnp.float32),
    pltpu.VMEM((CHUNK, DIM), jnp.float32),
    pltpu.VMEM_SHARED((N_PAD, DIM), jnp.float32),
    pltpu.SemaphoreType.DMA,
    pltpu.SemaphoreType.DMA,
    pltpu.SemaphoreType.DMA,
    pltpu.SemaphoreType.DMA,
]


def _edge_scatter_phase(g_hbm, src_hbm, dst_hbm, zeros_hbm, sidx_v,
                        didx_a, didx_b, didx_t, rows_a, rows_b, acc_sh,
                        sem_a, sem_b, sem_da, sem_db, s, wid):
    ebase = wid * EPW

    pltpu.sync_copy(zeros_hbm, rows_a)
    zdescs = [
        pltpu.async_copy(
            rows_a,
            acc_sh.at[pl.ds(s * ROWS_PER_TILE + t * OUT_STEP, OUT_STEP)],
            sem_a)
        for t in range(NOUT)
    ]
    pltpu.sync_copy(src_hbm.at[pl.ds(ebase, EPW)], sidx_v)
    for d in zdescs:
        d.wait()

    plsc.subcore_barrier()

    def gidx(j):
        return sidx_v.at[pl.ds(pl.multiple_of(j * CHUNK, 8), CHUNK)]

    def dsl(j):
        return dst_hbm.at[pl.ds(pl.multiple_of(ebase + j * CHUNK, 8), CHUNK)]

    pltpu.async_copy(g_hbm.at[gidx(0)], rows_a, sem_a)
    pltpu.async_copy(dsl(0), didx_a, sem_da)

    def edge_body(i, carry):
        j = 2 * i
        pltpu.async_copy(g_hbm.at[gidx(j + 1)], rows_b, sem_b)
        pltpu.async_copy(dsl(j + 1), didx_b, sem_db)
        pltpu.make_async_copy(g_hbm.at[gidx(0)], rows_a, sem_a).wait()
        pltpu.make_async_copy(dsl(0), didx_a, sem_da).wait()
        pltpu.sync_copy(rows_a, acc_sh.at[didx_a], add=True)

        @pl.when(j + 2 < NFULL)
        def _():
            pltpu.async_copy(g_hbm.at[gidx(j + 2)], rows_a, sem_a)
            pltpu.async_copy(dsl(j + 2), didx_a, sem_da)

        pltpu.make_async_copy(g_hbm.at[gidx(0)], rows_b, sem_b).wait()
        pltpu.make_async_copy(dsl(0), didx_b, sem_db).wait()
        pltpu.sync_copy(rows_b, acc_sh.at[didx_b], add=True)
        return carry

    lax.fori_loop(0, NFULL // 2, edge_body, 0)

    toff = pl.multiple_of(ebase + NFULL * CHUNK, 8)
    pltpu.sync_copy(dst_hbm.at[pl.ds(toff, TAIL)], didx_t)
    trows = rows_a.at[pl.ds(0, TAIL)]
    tidx = sidx_v.at[pl.ds(pl.multiple_of(NFULL * CHUNK, 8), TAIL)]
    pltpu.async_copy(g_hbm.at[tidx], trows, sem_a).wait()
    pltpu.sync_copy(trows, acc_sh.at[didx_t], add=True)

    plsc.subcore_barrier()


@functools.partial(
    pl.kernel,
    out_type=(
        jax.ShapeDtypeStruct((N_PAD, DIM), jnp.float32),
        jax.ShapeDtypeStruct((N_PAD, DIM), jnp.float32),
    ),
    mesh=_MESH,
    scratch_types=list(_AGG_SCRATCH),
    compiler_params=_SC_PARAMS,
)
def _agg_kernel(g_hbm, src_hbm, dst_hbm, zeros_hbm, out_a, out_b,
                sidx_v, didx_a, didx_b, didx_t, rows_a, rows_b, acc_sh,
                sem_a, sem_b, sem_da, sem_db):
    c = lax.axis_index("c")
    s = lax.axis_index("s")
    wid = s * NC + c

    _edge_scatter_phase(g_hbm, src_hbm, dst_hbm, zeros_hbm, sidx_v,
                        didx_a, didx_b, didx_t, rows_a, rows_b, acc_sh,
                        sem_a, sem_b, sem_da, sem_db, s, wid)

    odescs = [None] * NOUT
    for t in range(NOUT):
        full, sem_t = (rows_a, sem_a) if t % 2 == 0 else (rows_b, sem_b)
        buf = full.at[pl.ds(0, OUT_STEP)]
        if t >= 2:
            odescs[t - 2].wait()
        r = s * ROWS_PER_TILE + t * OUT_STEP
        pltpu.sync_copy(acc_sh.at[pl.ds(r, OUT_STEP)], buf)

        @pl.when(c == 0)
        def _():
            pltpu.async_copy(buf, out_a.at[pl.ds(r, OUT_STEP)], sem_t)

        @pl.when(c == 1)
        def _():
            pltpu.async_copy(buf, out_b.at[pl.ds(r, OUT_STEP)], sem_t)

        odescs[t] = pltpu.make_async_copy(
            buf, out_a.at[pl.ds(r, OUT_STEP)], sem_t)
    odescs[NOUT - 2].wait()
    odescs[NOUT - 1].wait()


@functools.partial(
    pl.kernel,
    out_type=(
        jax.ShapeDtypeStruct((BATCH, DIM), jnp.float32),
        jax.ShapeDtypeStruct((BATCH, DIM), jnp.float32),
        jax.ShapeDtypeStruct((BATCH, DIM), jnp.float32),
        jax.ShapeDtypeStruct((BATCH, DIM), jnp.float32),
    ),
    mesh=_MESH,
    scratch_types=list(_AGG_SCRATCH) + [pltpu.VMEM((BPT,), jnp.int32)],
    compiler_params=_SC_PARAMS,
)
def _agg_gather_kernel(g_hbm, src_hbm, dst_hbm, zeros_hbm, ut, dinv128, uidx,
                       y1a, y1b, yu, yd,
                       sidx_v, didx_a, didx_b, didx_t, rows_a, rows_b, acc_sh,
                       sem_a, sem_b, sem_da, sem_db, uidx_v):
    c = lax.axis_index("c")
    s = lax.axis_index("s")
    wid = s * NC + c

    pltpu.sync_copy(uidx.at[pl.ds(s * BPT, BPT)], uidx_v)
    _edge_scatter_phase(g_hbm, src_hbm, dst_hbm, zeros_hbm, sidx_v,
                        didx_a, didx_b, didx_t, rows_a, rows_b, acc_sh,
                        sem_a, sem_b, sem_da, sem_db, s, wid)

    GSTEP = 64
    gbuf = rows_a.at[pl.ds(0, GSTEP)]
    for k in range(BPT // GSTEP):
        gslice = uidx_v.at[pl.ds(k * GSTEP, GSTEP)]
        pltpu.async_copy(acc_sh.at[gslice], gbuf, sem_a).wait()

        @pl.when(c == 0)
        def _():
            pltpu.async_copy(g_hbm.at[gslice], gbuf, sem_a, add=True).wait()
            pltpu.sync_copy(gbuf, y1a.at[pl.ds(s * BPT + k * GSTEP, GSTEP)])

        @pl.when(c == 1)
        def _():
            pltpu.sync_copy(gbuf, y1b.at[pl.ds(s * BPT + k * GSTEP, GSTEP)])

    hbuf = rows_b.at[pl.ds(0, GSTEP)]
    for tbl, outy in ((ut, yu), (dinv128, yd)):
        for k in range(BPW // GSTEP):
            off = c * BPW + k * GSTEP
            hslice = uidx_v.at[pl.ds(off, GSTEP)]
            pltpu.async_copy(tbl.at[hslice], hbuf, sem_b).wait()
            pltpu.sync_copy(
                hbuf, outy.at[pl.ds(wid * BPW + k * GSTEP, GSTEP)])


def _dense1_body(ut_ref, poi_ref, Wp_ref, bp_ref, Wg1_ref, degp_ref,
                 g1_ref, dinv16_ref, dinv128_ref):
    deg = jnp.sum(degp_ref[...], axis=0) + 1.0
    dinv = lax.rsqrt(deg)
    hp = jnp.dot(poi_ref[...], Wp_ref[...],
                 preferred_element_type=jnp.float32) + bp_ref[...]
    h1u = jnp.dot(ut_ref[...], Wg1_ref[...],
                  preferred_element_type=jnp.float32)
    h1p = jnp.dot(hp, Wg1_ref[...], preferred_element_type=jnp.float32)
    g1_ref[0:NUM_USERS, :] = h1u * dinv[0:NUM_USERS, None]
    g1_ref[NUM_USERS:N_NODES, :] = h1p * dinv[NUM_USERS:N_NODES, None]
    g1_ref[N_NODES:N_PAD, :] = jnp.zeros((N_PAD - N_NODES, DIM), jnp.float32)
    dinv16_ref[0:N_NODES, :] = jnp.broadcast_to(dinv[:, None],
                                                (N_NODES, LANES))
    dinv16_ref[N_NODES:N_PAD, :] = jnp.ones((N_PAD - N_NODES, LANES),
                                            jnp.float32)
    dinv128_ref[0:N_NODES, :] = jnp.broadcast_to(dinv[:, None],
                                                 (N_NODES, DIM))
    dinv128_ref[N_NODES:N_PAD, :] = jnp.ones((N_PAD - N_NODES, DIM),
                                             jnp.float32)


_dense1 = pl.pallas_call(
    _dense1_body,
    out_shape=(
        jax.ShapeDtypeStruct((N_PAD, DIM), jnp.float32),
        jax.ShapeDtypeStruct((N_PAD, LANES), jnp.float32),
        jax.ShapeDtypeStruct((N_PAD, DIM), jnp.float32),
    ),
)


def _dense2_body(sa_ref, sb_ref, g1_ref, dinv16_ref, bg1_ref, Wg2_ref,
                 g2_ref):
    dinv = dinv16_ref[:, 0:1]
    t = dinv * (sa_ref[...] + sb_ref[...] + g1_ref[...]) + bg1_ref[...]
    x1 = jnp.maximum(t, 0.2 * t)
    g2_ref[...] = jnp.dot(x1, Wg2_ref[...],
                          preferred_element_type=jnp.float32) * dinv


_dense2 = pl.pallas_call(
    _dense2_body,
    out_shape=jax.ShapeDtypeStruct((N_PAD, DIM), jnp.float32),
)


def _final_body(y1a_ref, y1b_ref, yu_ref, yd_ref, bg2_ref, Wf_ref, bf_ref,
                out_ref):
    t = yd_ref[...] * (y1a_ref[...] + y1b_ref[...]) + bg2_ref[...]
    x2 = jnp.maximum(t, 0.2 * t)
    out_ref[...] = jnp.dot(x2 + yu_ref[...], Wf_ref[...],
                           preferred_element_type=jnp.float32) + bf_ref[...]


_final = pl.pallas_call(
    _final_body,
    out_shape=jax.ShapeDtypeStruct((BATCH, DIM), jnp.float32),
)


@jax.jit
def kernel(user_idx, poi_embeddings, edge_index, user_table,
           Wp, bp, Wg1, bg1, Wg2, bg2, Wf, bf):
    src = edge_index[0].astype(jnp.int32)
    dst = edge_index[1].astype(jnp.int32)
    zeros = jnp.zeros((OUT_STEP, DIM), jnp.float32)
    uidx = user_idx.astype(jnp.int32)

    degp = _deg_kernel(dst).reshape(NW, N_NODES)
    g1, dinv16, dinv128 = _dense1(user_table, poi_embeddings, Wp, bp, Wg1,
                                  degp)
    s1a, s1b = _agg_kernel(g1, src, dst, zeros)
    g2 = _dense2(s1a, s1b, g1, dinv16, bg1, Wg2)
    y1a, y1b, yu, yd = _agg_gather_kernel(g2, src, dst, zeros,
                                          user_table, dinv128, uidx)
    return _final(y1a, y1b, yu, yd, bg2, Wf, bf)


# speedup vs baseline: 1.0290x; 1.0001x over previous
import functools

import jax
import jax.numpy as jnp
from jax import lax
from jax.experimental import pallas as pl
from jax.experimental.pallas import tpu as pltpu
from jax.experimental.pallas import tpu_sc as plsc

NUM_USERS = 8000
NUM_POIS = 2000
N_NODES = 10000
N_PAD = 10240
DIM = 128
POI_DIM = 256
N_EDGES = 320000
BATCH = 4096

NC = 2
NS = 16
NW = NC * NS
LANES = 16

EPW = N_EDGES // NW
CHUNK = 128
NFULL = EPW // CHUNK
TAIL = EPW - NFULL * CHUNK
ROWS_PER_TILE = N_PAD // NS
OUT_STEP = 128
NOUT = ROWS_PER_TILE // OUT_STEP
BPW = BATCH // NW
BPT = BATCH // NS

_MESH = plsc.VectorSubcoreMesh(core_axis_name="c", subcore_axis_name="s")
_SC_PARAMS = pltpu.CompilerParams(needs_layout_passes=False)


@functools.partial(
    pl.kernel,
    out_type=jax.ShapeDtypeStruct((NW * N_NODES,), jnp.float32),
    mesh=_MESH,
    scratch_types=[
        pltpu.VMEM((EPW,), jnp.int32),
        pltpu.VMEM((N_NODES,), jnp.float32),
    ],
    compiler_params=_SC_PARAMS,
)
def _deg_kernel(dst_hbm, degp_hbm, idx_v, deg_v):
    c = lax.axis_index("c")
    s = lax.axis_index("s")
    wid = s * NC + c

    zeros16 = jnp.zeros((LANES,), jnp.float32)

    def zero_body(i, carry):
        deg_v[pl.ds(i * LANES, LANES)] = zeros16
        return carry

    lax.fori_loop(0, N_NODES // LANES, zero_body, 0)

    pltpu.sync_copy(dst_hbm.at[pl.ds(wid * EPW, EPW)], idx_v)

    ones16 = jnp.ones((LANES,), jnp.float32)

    def hist_body(i, carry):
        idx = idx_v[pl.ds(i * LANES, LANES)]
        plsc.addupdate_scatter(deg_v, [idx], ones16)
        return carry

    lax.fori_loop(0, EPW // LANES, hist_body, 0)

    pltpu.sync_copy(deg_v, degp_hbm.at[pl.ds(wid * N_NODES, N_NODES)])


_AGG_SCRATCH = [
    pltpu.VMEM((EPW,), jnp.int32),
    pltpu.VMEM((CHUNK,), jnp.int32),
    pltpu.VMEM((CHUNK,), jnp.int32),
    pltpu.VMEM((TAIL,), jnp.int32),
    pltpu.VMEM((CHUNK, DIM), jnp.float32),
    pltpu.VMEM((CHUNK, DIM), jnp.float32),
    pltpu.VMEM_SHARED((N_PAD, DIM), jnp.float32),
    pltpu.SemaphoreType.DMA,
    pltpu.SemaphoreType.DMA,
    pltpu.SemaphoreType.DMA,
    pltpu.SemaphoreType.DMA,
]


def _edge_scatter_phase(g_hbm, src_hbm, dst_hbm, zeros_hbm, sidx_v,
                        didx_a, didx_b, didx_t, rows_a, rows_b, acc_sh,
                        sem_a, sem_b, sem_da, sem_db, s, wid):
    ebase = wid * EPW

    pltpu.sync_copy(zeros_hbm, rows_a)
    zdescs = [
        pltpu.async_copy(
            rows_a,
            acc_sh.at[pl.ds(s * ROWS_PER_TILE + t * OUT_STEP, OUT_STEP)],
            sem_a)
        for t in range(NOUT)
    ]
    pltpu.sync_copy(src_hbm.at[pl.ds(ebase, EPW)], sidx_v)
    for d in zdescs:
        d.wait()

    def gidx(j):
        return sidx_v.at[pl.ds(pl.multiple_of(j * CHUNK, 8), CHUNK)]

    def dsl(j):
        return dst_hbm.at[pl.ds(pl.multiple_of(ebase + j * CHUNK, 8), CHUNK)]

    pltpu.async_copy(g_hbm.at[gidx(0)], rows_a, sem_a)
    pltpu.async_copy(dsl(0), didx_a, sem_da)
    pltpu.async_copy(g_hbm.at[gidx(1)], rows_b, sem_b)
    pltpu.async_copy(dsl(1), didx_b, sem_db)

    plsc.subcore_barrier()

    def edge_body(i, carry):
        j = 2 * i
        pltpu.make_async_copy(g_hbm.at[gidx(0)], rows_a, sem_a).wait()
        pltpu.make_async_copy(dsl(0), didx_a, sem_da).wait()
        pltpu.sync_copy(rows_a, acc_sh.at[didx_a], add=True)

        @pl.when(j + 2 < NFULL)
        def _():
            pltpu.async_copy(g_hbm.at[gidx(j + 2)], rows_a, sem_a)
            pltpu.async_copy(dsl(j + 2), didx_a, sem_da)

        pltpu.make_async_copy(g_hbm.at[gidx(0)], rows_b, sem_b).wait()
        pltpu.make_async_copy(dsl(0), didx_b, sem_db).wait()
        pltpu.sync_copy(rows_b, acc_sh.at[didx_b], add=True)

        @pl.when(j + 3 < NFULL)
        def _():
            pltpu.async_copy(g_hbm.at[gidx(j + 3)], rows_b, sem_b)
            pltpu.async_copy(dsl(j + 3), didx_b, sem_db)

        return carry

    lax.fori_loop(0, NFULL // 2, edge_body, 0)

    toff = pl.multiple_of(ebase + NFULL * CHUNK, 8)
    pltpu.sync_copy(dst_hbm.at[pl.ds(toff, TAIL)], didx_t)
    trows = rows_a.at[pl.ds(0, TAIL)]
    tidx = sidx_v.at[pl.ds(pl.multiple_of(NFULL * CHUNK, 8), TAIL)]
    pltpu.async_copy(g_hbm.at[tidx], trows, sem_a).wait()
    pltpu.sync_copy(trows, acc_sh.at[didx_t], add=True)

    plsc.subcore_barrier()


@functools.partial(
    pl.kernel,
    out_type=(
        jax.ShapeDtypeStruct((N_PAD, DIM), jnp.float32),
        jax.ShapeDtypeStruct((N_PAD, DIM), jnp.float32),
    ),
    mesh=_MESH,
    scratch_types=list(_AGG_SCRATCH),
    compiler_params=_SC_PARAMS,
)
def _agg_kernel(g_hbm, src_hbm, dst_hbm, zeros_hbm, out_a, out_b,
                sidx_v, didx_a, didx_b, didx_t, rows_a, rows_b, acc_sh,
                sem_a, sem_b, sem_da, sem_db):
    c = lax.axis_index("c")
    s = lax.axis_index("s")
    wid = s * NC + c

    _edge_scatter_phase(g_hbm, src_hbm, dst_hbm, zeros_hbm, sidx_v,
                        didx_a, didx_b, didx_t, rows_a, rows_b, acc_sh,
                        sem_a, sem_b, sem_da, sem_db, s, wid)

    odescs = [None] * NOUT
    for t in range(NOUT):
        full, sem_t = (rows_a, sem_a) if t % 2 == 0 else (rows_b, sem_b)
        buf = full.at[pl.ds(0, OUT_STEP)]
        if t >= 2:
            odescs[t - 2].wait()
        r = s * ROWS_PER_TILE + t * OUT_STEP
        pltpu.sync_copy(acc_sh.at[pl.ds(r, OUT_STEP)], buf)

        @pl.when(c == 0)
        def _():
            pltpu.async_copy(buf, out_a.at[pl.ds(r, OUT_STEP)], sem_t)

        @pl.when(c == 1)
        def _():
            pltpu.async_copy(buf, out_b.at[pl.ds(r, OUT_STEP)], sem_t)

        odescs[t] = pltpu.make_async_copy(
            buf, out_a.at[pl.ds(r, OUT_STEP)], sem_t)
    odescs[NOUT - 2].wait()
    odescs[NOUT - 1].wait()


@functools.partial(
    pl.kernel,
    out_type=(
        jax.ShapeDtypeStruct((BATCH, DIM), jnp.float32),
        jax.ShapeDtypeStruct((BATCH, DIM), jnp.float32),
        jax.ShapeDtypeStruct((BATCH, DIM), jnp.float32),
        jax.ShapeDtypeStruct((BATCH, DIM), jnp.float32),
    ),
    mesh=_MESH,
    scratch_types=list(_AGG_SCRATCH) + [pltpu.VMEM((BPT,), jnp.int32)],
    compiler_params=_SC_PARAMS,
)
def _agg_gather_kernel(g_hbm, src_hbm, dst_hbm, zeros_hbm, ut, dinv128, uidx,
                       y1a, y1b, yu, yd,
                       sidx_v, didx_a, didx_b, didx_t, rows_a, rows_b, acc_sh,
                       sem_a, sem_b, sem_da, sem_db, uidx_v):
    c = lax.axis_index("c")
    s = lax.axis_index("s")
    wid = s * NC + c

    pltpu.sync_copy(uidx.at[pl.ds(s * BPT, BPT)], uidx_v)
    _edge_scatter_phase(g_hbm, src_hbm, dst_hbm, zeros_hbm, sidx_v,
                        didx_a, didx_b, didx_t, rows_a, rows_b, acc_sh,
                        sem_a, sem_b, sem_da, sem_db, s, wid)

    GSTEP = 64
    gbuf = rows_a.at[pl.ds(0, GSTEP)]
    for k in range(BPT // GSTEP):
        gslice = uidx_v.at[pl.ds(k * GSTEP, GSTEP)]
        pltpu.async_copy(acc_sh.at[gslice], gbuf, sem_a).wait()

        @pl.when(c == 0)
        def _():
            pltpu.async_copy(g_hbm.at[gslice], gbuf, sem_a, add=True).wait()
            pltpu.sync_copy(gbuf, y1a.at[pl.ds(s * BPT + k * GSTEP, GSTEP)])

        @pl.when(c == 1)
        def _():
            pltpu.sync_copy(gbuf, y1b.at[pl.ds(s * BPT + k * GSTEP, GSTEP)])

    hbuf = rows_b.at[pl.ds(0, GSTEP)]
    for tbl, outy in ((ut, yu), (dinv128, yd)):
        for k in range(BPW // GSTEP):
            off = c * BPW + k * GSTEP
            hslice = uidx_v.at[pl.ds(off, GSTEP)]
            pltpu.async_copy(tbl.at[hslice], hbuf, sem_b).wait()
            pltpu.sync_copy(
                hbuf, outy.at[pl.ds(wid * BPW + k * GSTEP, GSTEP)])


def _dense1_body(ut_ref, poi_ref, Wp_ref, bp_ref, Wg1_ref, degp_ref,
                 g1_ref, dinv16_ref, dinv128_ref):
    deg = jnp.sum(degp_ref[...], axis=0) + 1.0
    dinv = lax.rsqrt(deg)
    hp = jnp.dot(poi_ref[...], Wp_ref[...],
                 preferred_element_type=jnp.float32) + bp_ref[...]
    h1u = jnp.dot(ut_ref[...], Wg1_ref[...],
                  preferred_element_type=jnp.float32)
    h1p = jnp.dot(hp, Wg1_ref[...], preferred_element_type=jnp.float32)
    g1_ref[0:NUM_USERS, :] = h1u * dinv[0:NUM_USERS, None]
    g1_ref[NUM_USERS:N_NODES, :] = h1p * dinv[NUM_USERS:N_NODES, None]
    g1_ref[N_NODES:N_PAD, :] = jnp.zeros((N_PAD - N_NODES, DIM), jnp.float32)
    dinv16_ref[0:N_NODES, :] = jnp.broadcast_to(dinv[:, None],
                                                (N_NODES, LANES))
    dinv16_ref[N_NODES:N_PAD, :] = jnp.ones((N_PAD - N_NODES, LANES),
                                            jnp.float32)
    dinv128_ref[0:N_NODES, :] = jnp.broadcast_to(dinv[:, None],
                                                 (N_NODES, DIM))
    dinv128_ref[N_NODES:N_PAD, :] = jnp.ones((N_PAD - N_NODES, DIM),
                                             jnp.float32)


_dense1 = pl.pallas_call(
    _dense1_body,
    out_shape=(
        jax.ShapeDtypeStruct((N_PAD, DIM), jnp.float32),
        jax.ShapeDtypeStruct((N_PAD, LANES), jnp.float32),
        jax.ShapeDtypeStruct((N_PAD, DIM), jnp.float32),
    ),
)


def _dense2_body(sa_ref, sb_ref, g1_ref, dinv16_ref, bg1_ref, Wg2_ref,
                 g2_ref):
    dinv = dinv16_ref[:, 0:1]
    t = dinv * (sa_ref[...] + sb_ref[...] + g1_ref[...]) + bg1_ref[...]
    x1 = jnp.maximum(t, 0.2 * t)
    g2_ref[...] = jnp.dot(x1, Wg2_ref[...],
                          preferred_element_type=jnp.float32) * dinv


_dense2 = pl.pallas_call(
    _dense2_body,
    out_shape=jax.ShapeDtypeStruct((N_PAD, DIM), jnp.float32),
)


def _final_body(y1a_ref, y1b_ref, yu_ref, yd_ref, bg2_ref, Wf_ref, bf_ref,
                out_ref):
    t = yd_ref[...] * (y1a_ref[...] + y1b_ref[...]) + bg2_ref[...]
    x2 = jnp.maximum(t, 0.2 * t)
    out_ref[...] = jnp.dot(x2 + yu_ref[...], Wf_ref[...],
                           preferred_element_type=jnp.float32) + bf_ref[...]


_final = pl.pallas_call(
    _final_body,
    out_shape=jax.ShapeDtypeStruct((BATCH, DIM), jnp.float32),
)


@jax.jit
def kernel(user_idx, poi_embeddings, edge_index, user_table,
           Wp, bp, Wg1, bg1, Wg2, bg2, Wf, bf):
    src = edge_index[0].astype(jnp.int32)
    dst = edge_index[1].astype(jnp.int32)
    zeros = jnp.zeros((OUT_STEP, DIM), jnp.float32)
    uidx = user_idx.astype(jnp.int32)

    degp = _deg_kernel(dst).reshape(NW, N_NODES)
    g1, dinv16, dinv128 = _dense1(user_table, poi_embeddings, Wp, bp, Wg1,
                                  degp)
    s1a, s1b = _agg_kernel(g1, src, dst, zeros)
    g2 = _dense2(s1a, s1b, g1, dinv16, bg1, Wg2)
    y1a, y1b, yu, yd = _agg_gather_kernel(g2, src, dst, zeros,
                                          user_table, dinv128, uidx)
    return _final(y1a, y1b, yu, yd, bg2, Wf, bf)
